# baseline (speedup 1.0000x reference)
import functools

import jax
import jax.numpy as jnp
from jax import lax
from jax.experimental import pallas as pl
from jax.experimental.pallas import tpu as pltpu
from jax.experimental.pallas import tpu_sc as plsc

_B, _FU, _FI = 16384, 26, 20
_D = 64
_UV, _IV = 100000, 1000
_NC, _NS = 2, 16
_NW = _NC * _NS
_EPW = _B // _NW
_CHUNK = 8
_NCHUNK = _EPW // _CHUNK
_UROW = _CHUNK * _FU
_IROW = _CHUNK * _FI


def _renorm_body(x_ref, o_ref):
    halves = []
    for h in range(2):
        x = x_ref[h]
        norm = jnp.sqrt(jnp.sum(x * x, axis=1, keepdims=True))
        scale = jnp.where(norm > 1.0, 1.0 / (norm + 1e-7), 1.0)
        halves.append(x * scale)
    o_ref[...] = jnp.concatenate(halves, axis=1)


def _renorm_table(table, blk):
    v, d = table.shape
    nblk = v // 2 // blk
    return pl.pallas_call(
        _renorm_body,
        grid=(nblk,),
        in_specs=[pl.BlockSpec((2, blk, d), lambda g: (0, g, 0))],
        out_specs=pl.BlockSpec((blk, 2 * d), lambda g: (g, 0)),
        out_shape=jax.ShapeDtypeStruct((v // 2, 2 * d), table.dtype),
    )(table.reshape(2, v // 2, d))


def _renorm_plain_body(x_ref, o_ref):
    x = x_ref[...]
    norm = jnp.sqrt(jnp.sum(x * x, axis=1, keepdims=True))
    scale = jnp.where(norm > 1.0, 1.0 / (norm + 1e-7), 1.0)
    o_ref[...] = x * scale


def _renorm_plain(table):
    return pl.pallas_call(
        _renorm_plain_body,
        out_shape=jax.ShapeDtypeStruct(table.shape, table.dtype),
    )(table)


def _lane_permute(x, idx):
    return lax.gather(
        x, idx[:, None],
        lax.GatherDimensionNumbers(offset_dims=(), collapsed_slice_dims=(0,),
                                   start_index_map=(0,)),
        slice_sizes=(1,),
        mode=lax.GatherScatterMode.PROMISE_IN_BOUNDS)


def _fm_body(u2_hbm, i2_hbm, ut_hbm, it_hbm, out_hbm,
             u2d, uflat, iidx, urows, irows, outbuf, itsp, usem, isem):
    wid = lax.axis_index("s") * _NC + lax.axis_index("c")
    pltpu.sync_copy(u2_hbm.at[pl.ds(wid * _EPW, _EPW), :], u2d)
    pltpu.sync_copy(i2_hbm.at[pl.ds(wid * _EPW, _EPW), :], iidx)
    lanes = lax.iota(jnp.int32, 16)

    @pl.when(lax.axis_index("s") == 0)
    def _():
        pltpu.sync_copy(it_hbm, itsp)

    plsc.subcore_barrier()

    def remap(t):
        return jnp.where(t >= _UV // 2, 2 * t - (_UV - 1), 2 * t)

    def remap_body(r, carry):
        uflat[pl.ds(r * _FU, 16)] = remap(u2d[r, pl.ds(0, 16)])
        uflat[pl.ds(r * _FU + _FU - 16, 16)] = remap(u2d[r, pl.ds(_FU - 16, 16)])
        return carry

    lax.fori_loop(0, _EPW, remap_body, 0)

    def fire(c, b):
        for j in range(_UROW // 104):
            pltpu.async_copy(
                ut_hbm.at[uflat.at[pl.ds(c * _UROW + j * 104, 104)]],
                urows.at[b, pl.ds(j * 104, 104), :], usem)
        for e in range(_CHUNK):
            pltpu.async_copy(itsp.at[iidx.at[c * _CHUNK + e]],
                             irows.at[b, pl.ds(e * _FI, _FI), :], isem)

    def drain(b):
        pltpu.make_async_copy(ut_hbm.at[pl.ds(0, _UROW), :],
                              urows.at[b], usem).wait()
        pltpu.make_async_copy(itsp.at[pl.ds(0, _IROW), :],
                              irows.at[b], isem).wait()

    def process(c, b, off, init):
        def elem_body(e, outv):
            a = [jnp.zeros((16,), jnp.float32) for _ in range(4)]
            q = [jnp.zeros((16,), jnp.float32) for _ in range(4)]
            s = [jnp.zeros((16,), jnp.float32) for _ in range(4)]
            for f in range(_FU):
                r = e * _FU + f
                for k in range(4):
                    row = urows[b, r, pl.ds(k * 16, 16)]
                    a[k] = a[k] + row
                    q[k] = q[k] + row * row
            for f in range(_FI):
                r = e * _FI + f
                for k in range(4):
                    row = irows[b, r, pl.ds(k * 16, 16)]
                    s[k] = s[k] + row
            acc = jnp.zeros((16,), jnp.float32)
            for k in range(4):
                t = a[k] + s[k]
                acc = acc + (t * t - q[k] - s[k] * s[k])
            for shift in (1, 2, 4, 8):
                acc = acc + _lane_permute(acc, lanes ^ shift)
            return jnp.where(lanes == e + off, acc, outv)

        return lax.fori_loop(0, _CHUNK, elem_body, init)

    fire(0, 0)

    def pipe_body(it_, carry):
        c = 2 * it_

        @pl.when(c + 1 < _NCHUNK)
        def _():
            fire(c + 1, 1)

        drain(0)
        o0 = process(c, 0, 0, jnp.zeros((16,), jnp.float32))

        @pl.when(c + 2 < _NCHUNK)
        def _():
            fire(c + 2, 0)

        drain(1)
        outv = process(c + 1, 1, _CHUNK, o0)
        outbuf[pl.ds(c * _CHUNK, 16)] = 1.0 / (1.0 + jnp.exp(-0.5 * outv))
        return carry

    lax.fori_loop(0, _NCHUNK // 2, pipe_body, 0)
    pltpu.sync_copy(outbuf, out_hbm.at[pl.ds(wid * _EPW, _EPW)])


_fm_sc = functools.partial(
    pl.kernel,
    out_type=jax.ShapeDtypeStruct((_B,), jnp.float32),
    mesh=plsc.VectorSubcoreMesh(core_axis_name="c", subcore_axis_name="s"),
    scratch_types=[
        pltpu.VMEM((_EPW, _FU), jnp.int32),
        pltpu.VMEM((_EPW * _FU,), jnp.int32),
        pltpu.VMEM((_EPW, _FI), jnp.int32),
        pltpu.VMEM((2, _UROW, _D), jnp.float32),
        pltpu.VMEM((2, _IROW, _D), jnp.float32),
        pltpu.VMEM((_EPW,), jnp.float32),
        pltpu.VMEM_SHARED((_IV, _D), jnp.float32),
        pltpu.SemaphoreType.DMA,
        pltpu.SemaphoreType.DMA,
    ],
    compiler_params=pltpu.CompilerParams(use_tc_tiling_on_sc=False),
)(_fm_body)


def kernel(u, i, user_table, item_table):
    ut = _renorm_table(user_table, 10000)
    it = _renorm_plain(item_table)
    out = _fm_sc(u.astype(jnp.int32), i.astype(jnp.int32),
                 ut.reshape(_UV, _D), it)
    return out.reshape(_B, 1)


# speedup vs baseline: 1.1477x; 1.1477x over previous
import functools

import jax
import jax.numpy as jnp
from jax import lax
from jax.experimental import pallas as pl
from jax.experimental.pallas import tpu as pltpu
from jax.experimental.pallas import tpu_sc as plsc

_B, _FU, _FI = 16384, 26, 20
_D = 64
_UV, _IV = 100000, 1000
_NC, _NS = 2, 16
_NW = _NC * _NS
_EPW = _B // _NW
_CHUNK = 16
_NCHUNK = _EPW // _CHUNK
_UROW = _CHUNK * _FU
_IROW = _CHUNK * _FI


def _renorm_body(x_ref, o_ref):
    halves = []
    for h in range(2):
        x = x_ref[h]
        norm = jnp.sqrt(jnp.sum(x * x, axis=1, keepdims=True))
        scale = jnp.where(norm > 1.0, 1.0 / (norm + 1e-7), 1.0)
        halves.append(x * scale)
    o_ref[...] = jnp.concatenate(halves, axis=1)


def _renorm_table(table, blk):
    v, d = table.shape
    nblk = v // 2 // blk
    return pl.pallas_call(
        _renorm_body,
        grid=(nblk,),
        in_specs=[pl.BlockSpec((2, blk, d), lambda g: (0, g, 0))],
        out_specs=pl.BlockSpec((blk, 2 * d), lambda g: (g, 0)),
        out_shape=jax.ShapeDtypeStruct((v // 2, 2 * d), table.dtype),
    )(table.reshape(2, v // 2, d))


def _renorm_plain_body(x_ref, o_ref):
    x = x_ref[...]
    norm = jnp.sqrt(jnp.sum(x * x, axis=1, keepdims=True))
    scale = jnp.where(norm > 1.0, 1.0 / (norm + 1e-7), 1.0)
    o_ref[...] = x * scale


def _renorm_plain(table):
    return pl.pallas_call(
        _renorm_plain_body,
        out_shape=jax.ShapeDtypeStruct(table.shape, table.dtype),
    )(table)


def _lane_permute(x, idx):
    return lax.gather(
        x, idx[:, None],
        lax.GatherDimensionNumbers(offset_dims=(), collapsed_slice_dims=(0,),
                                   start_index_map=(0,)),
        slice_sizes=(1,),
        mode=lax.GatherScatterMode.PROMISE_IN_BOUNDS)


def _fm_body(u1_hbm, i1_hbm, ut_hbm, it_hbm, out_hbm,
             uidx, iidx, urows, irows, outbuf, itsp, usem, isem):
    wid = lax.axis_index("s") * _NC + lax.axis_index("c")
    pltpu.sync_copy(u1_hbm.at[pl.ds(wid * _EPW * _FU, _EPW * _FU)], uidx)
    pltpu.sync_copy(i1_hbm.at[pl.ds(wid * _EPW * _FI, _EPW * _FI)], iidx)
    lanes = lax.iota(jnp.int32, 16)

    @pl.when(lax.axis_index("s") == 0)
    def _():
        pltpu.sync_copy(it_hbm, itsp)

    plsc.subcore_barrier()

    def remap_body(j, carry):
        t = uidx[pl.ds(j * 16, 16)]
        uidx[pl.ds(j * 16, 16)] = jnp.where(
            t >= _UV // 2, 2 * t - (_UV - 1), 2 * t)
        return carry

    lax.fori_loop(0, _EPW * _FU // 16, remap_body, 0)

    def fire(c, b):
        for j in range(4):
            pltpu.async_copy(
                ut_hbm.at[uidx.at[pl.ds(c * _UROW + j * 104, 104)]],
                urows.at[b, pl.ds(j * 104, 104), :], usem)
            pltpu.async_copy(
                itsp.at[iidx.at[pl.ds(c * _IROW + j * 80, 80)]],
                irows.at[b, pl.ds(j * 80, 80), :], isem)

    def drain(b):
        pltpu.make_async_copy(ut_hbm.at[pl.ds(0, _UROW), :],
                              urows.at[b], usem).wait()
        pltpu.make_async_copy(itsp.at[pl.ds(0, _IROW), :],
                              irows.at[b], isem).wait()

    def process(c, b, off, init):
        def elem_body(e, outv):
            a = [jnp.zeros((16,), jnp.float32) for _ in range(4)]
            q = [jnp.zeros((16,), jnp.float32) for _ in range(4)]
            s = [jnp.zeros((16,), jnp.float32) for _ in range(4)]
            for f in range(_FU):
                r = e * _FU + f
                for k in range(4):
                    row = urows[b, r, pl.ds(k * 16, 16)]
                    a[k] = a[k] + row
                    q[k] = q[k] + row * row
            for f in range(_FI):
                r = e * _FI + f
                for k in range(4):
                    row = irows[b, r, pl.ds(k * 16, 16)]
                    s[k] = s[k] + row
            acc = jnp.zeros((16,), jnp.float32)
            for k in range(4):
                t = a[k] + s[k]
                acc = acc + (t * t - q[k] - s[k] * s[k])
            for shift in (1, 2, 4, 8):
                acc = acc + _lane_permute(acc, lanes ^ shift)
            return jnp.where(lanes == e + off, acc, outv)

        return lax.fori_loop(0, _CHUNK, elem_body, init)

    fire(0, 0)

    def pipe_body(it_, carry):
        c = 2 * it_

        @pl.when(c + 1 < _NCHUNK)
        def _():
            fire(c + 1, 1)

        drain(0)
        o0 = process(c, 0, 0, jnp.zeros((16,), jnp.float32))
        outbuf[pl.ds(c * _CHUNK, 16)] = 1.0 / (1.0 + jnp.exp(-0.5 * o0))

        @pl.when(c + 2 < _NCHUNK)
        def _():
            fire(c + 2, 0)

        drain(1)
        o1 = process(c + 1, 1, 0, jnp.zeros((16,), jnp.float32))
        outbuf[pl.ds((c + 1) * _CHUNK, 16)] = 1.0 / (1.0 + jnp.exp(-0.5 * o1))
        return carry

    lax.fori_loop(0, _NCHUNK // 2, pipe_body, 0)
    pltpu.sync_copy(outbuf, out_hbm.at[pl.ds(wid * _EPW, _EPW)])


_fm_sc = functools.partial(
    pl.kernel,
    out_type=jax.ShapeDtypeStruct((_B,), jnp.float32),
    mesh=plsc.VectorSubcoreMesh(core_axis_name="c", subcore_axis_name="s"),
    scratch_types=[
        pltpu.VMEM((_EPW * _FU,), jnp.int32),
        pltpu.VMEM((_EPW * _FI,), jnp.int32),
        pltpu.VMEM((2, _UROW, _D), jnp.float32),
        pltpu.VMEM((2, _IROW, _D), jnp.float32),
        pltpu.VMEM((_EPW,), jnp.float32),
        pltpu.VMEM_SHARED((_IV, _D), jnp.float32),
        pltpu.SemaphoreType.DMA,
        pltpu.SemaphoreType.DMA,
    ],
    compiler_params=pltpu.CompilerParams(use_tc_tiling_on_sc=False),
)(_fm_body)


def kernel(u, i, user_table, item_table):
    ut = _renorm_table(user_table, 10000)
    it = _renorm_plain(item_table)
    u1 = u.astype(jnp.int32).reshape(-1)
    i1 = i.astype(jnp.int32).reshape(-1)
    out = _fm_sc(u1, i1, ut.reshape(_UV, _D), it)
    return out.reshape(_B, 1)


# speedup vs baseline: 1.2477x; 1.0872x over previous
import functools

import jax
import jax.numpy as jnp
from jax import lax
from jax.experimental import pallas as pl
from jax.experimental.pallas import tpu as pltpu
from jax.experimental.pallas import tpu_sc as plsc

_B, _FU, _FI = 16384, 26, 20
_D = 64
_UV, _IV = 100000, 1000
_NC, _NS = 2, 16
_NW = _NC * _NS
_EPW = _B // _NW
_CHUNK = 16
_NCHUNK = _EPW // _CHUNK
_UROW = _CHUNK * _FU
_IROW = _CHUNK * _FI


def _renorm_body(x_ref, o_ref):
    halves = []
    for h in range(2):
        x = x_ref[h]
        norm = jnp.sqrt(jnp.sum(x * x, axis=1, keepdims=True))
        scale = jnp.where(norm > 1.0, 1.0 / (norm + 1e-7), 1.0)
        halves.append(x * scale)
    o_ref[...] = jnp.concatenate(halves, axis=1)


def _renorm_table(table, blk):
    v, d = table.shape
    nblk = v // 2 // blk
    return pl.pallas_call(
        _renorm_body,
        grid=(nblk,),
        in_specs=[pl.BlockSpec((2, blk, d), lambda g: (0, g, 0))],
        out_specs=pl.BlockSpec((blk, 2 * d), lambda g: (g, 0)),
        out_shape=jax.ShapeDtypeStruct((v // 2, 2 * d), table.dtype),
    )(table.reshape(2, v // 2, d))


def _renorm_plain_body(x_ref, o_ref):
    x = x_ref[...]
    norm = jnp.sqrt(jnp.sum(x * x, axis=1, keepdims=True))
    scale = jnp.where(norm > 1.0, 1.0 / (norm + 1e-7), 1.0)
    o_ref[...] = x * scale


def _renorm_plain(table):
    return pl.pallas_call(
        _renorm_plain_body,
        out_shape=jax.ShapeDtypeStruct(table.shape, table.dtype),
    )(table)


def _item_body(i1_hbm, it_hbm, sout_hbm, iidx, irows, sbuf, itsp, isem):
    wid = lax.axis_index("s") * _NC + lax.axis_index("c")
    pltpu.sync_copy(i1_hbm.at[pl.ds(wid * _EPW * _FI, _EPW * _FI)], iidx)

    @pl.when(lax.axis_index("s") == 0)
    def _():
        pltpu.sync_copy(it_hbm, itsp)

    plsc.subcore_barrier()

    def fire(c, b):
        for j in range(4):
            pltpu.async_copy(
                itsp.at[iidx.at[pl.ds(c * _IROW + j * 80, 80)]],
                irows.at[b, pl.ds(j * 80, 80), :], isem)

    def drain(b):
        pltpu.make_async_copy(itsp.at[pl.ds(0, _IROW), :],
                              irows.at[b], isem).wait()

    def process(c, b):
        def elem_body(e, carry):
            s = [jnp.zeros((16,), jnp.float32) for _ in range(4)]
            for f in range(_FI):
                r = e * _FI + f
                for k in range(4):
                    s[k] = s[k] + irows[b, r, pl.ds(k * 16, 16)]
            row = c * _CHUNK + e
            for k in range(4):
                sbuf[row, pl.ds(k * 16, 16)] = s[k]
            return carry

        lax.fori_loop(0, _CHUNK, elem_body, 0)

    fire(0, 0)

    def pipe_body(it_, carry):
        c = 2 * it_

        @pl.when(c + 1 < _NCHUNK)
        def _():
            fire(c + 1, 1)

        drain(0)
        process(c, 0)

        @pl.when(c + 2 < _NCHUNK)
        def _():
            fire(c + 2, 0)

        drain(1)
        process(c + 1, 1)
        return carry

    lax.fori_loop(0, _NCHUNK // 2, pipe_body, 0)
    pltpu.sync_copy(sbuf, sout_hbm.at[pl.ds(wid * _EPW, _EPW), :])


_item_sc = functools.partial(
    pl.kernel,
    out_type=jax.ShapeDtypeStruct((_B, _D), jnp.float32),
    mesh=plsc.VectorSubcoreMesh(core_axis_name="c", subcore_axis_name="s"),
    scratch_types=[
        pltpu.VMEM((_EPW * _FI,), jnp.int32),
        pltpu.VMEM((2, _IROW, _D), jnp.float32),
        pltpu.VMEM((_EPW, _D), jnp.float32),
        pltpu.VMEM_SHARED((_IV, _D), jnp.float32),
        pltpu.SemaphoreType.DMA,
    ],
    compiler_params=pltpu.CompilerParams(use_tc_tiling_on_sc=False),
)(_item_body)


def _lane_permute(x, idx):
    return lax.gather(
        x, idx[:, None],
        lax.GatherDimensionNumbers(offset_dims=(), collapsed_slice_dims=(0,),
                                   start_index_map=(0,)),
        slice_sizes=(1,),
        mode=lax.GatherScatterMode.PROMISE_IN_BOUNDS)


def _user_body(u1_hbm, s_hbm, ut_hbm, out_hbm,
               uidx, urows, srows, outbuf, usem, ssem):
    wid = lax.axis_index("s") * _NC + lax.axis_index("c")
    pltpu.sync_copy(u1_hbm.at[pl.ds(wid * _EPW * _FU, _EPW * _FU)], uidx)
    lanes = lax.iota(jnp.int32, 16)

    def remap_body(j, carry):
        t = uidx[pl.ds(j * 16, 16)]
        uidx[pl.ds(j * 16, 16)] = jnp.where(
            t >= _UV // 2, 2 * t - (_UV - 1), 2 * t)
        return carry

    lax.fori_loop(0, _EPW * _FU // 16, remap_body, 0)

    def fire(c, b):
        for j in range(4):
            pltpu.async_copy(
                ut_hbm.at[uidx.at[pl.ds(c * _UROW + j * 104, 104)]],
                urows.at[b, pl.ds(j * 104, 104), :], usem)
        pltpu.async_copy(s_hbm.at[pl.ds(wid * _EPW + c * _CHUNK, _CHUNK), :],
                         srows.at[b], ssem)

    def drain(c, b):
        pltpu.make_async_copy(ut_hbm.at[pl.ds(0, _UROW), :],
                              urows.at[b], usem).wait()
        pltpu.make_async_copy(s_hbm.at[pl.ds(0, _CHUNK), :],
                              srows.at[b], ssem).wait()

    def process(c, b):
        def elem_body(e, outv):
            a = [jnp.zeros((16,), jnp.float32) for _ in range(4)]
            q = [jnp.zeros((16,), jnp.float32) for _ in range(4)]
            for f in range(_FU):
                r = e * _FU + f
                for k in range(4):
                    row = urows[b, r, pl.ds(k * 16, 16)]
                    a[k] = a[k] + row
                    q[k] = q[k] + row * row
            acc = jnp.zeros((16,), jnp.float32)
            for k in range(4):
                s = srows[b, e, pl.ds(k * 16, 16)]
                t = a[k] + s
                acc = acc + (t * t - q[k] - s * s)
            for shift in (1, 2, 4, 8):
                acc = acc + _lane_permute(acc, lanes ^ shift)
            return jnp.where(lanes == e, acc, outv)

        return lax.fori_loop(0, _CHUNK, elem_body,
                             jnp.zeros((16,), jnp.float32))

    fire(0, 0)

    def pipe_body(it_, carry):
        c = 2 * it_

        @pl.when(c + 1 < _NCHUNK)
        def _():
            fire(c + 1, 1)

        drain(c, 0)
        o0 = process(c, 0)
        outbuf[pl.ds(c * _CHUNK, 16)] = 1.0 / (1.0 + jnp.exp(-0.5 * o0))

        @pl.when(c + 2 < _NCHUNK)
        def _():
            fire(c + 2, 0)

        drain(c + 1, 1)
        o1 = process(c + 1, 1)
        outbuf[pl.ds((c + 1) * _CHUNK, 16)] = 1.0 / (1.0 + jnp.exp(-0.5 * o1))
        return carry

    lax.fori_loop(0, _NCHUNK // 2, pipe_body, 0)
    pltpu.sync_copy(outbuf, out_hbm.at[pl.ds(wid * _EPW, _EPW)])


_user_sc = functools.partial(
    pl.kernel,
    out_type=jax.ShapeDtypeStruct((_B,), jnp.float32),
    mesh=plsc.VectorSubcoreMesh(core_axis_name="c", subcore_axis_name="s"),
    scratch_types=[
        pltpu.VMEM((_EPW * _FU,), jnp.int32),
        pltpu.VMEM((2, _UROW, _D), jnp.float32),
        pltpu.VMEM((2, _CHUNK, _D), jnp.float32),
        pltpu.VMEM((_EPW,), jnp.float32),
        pltpu.SemaphoreType.DMA,
        pltpu.SemaphoreType.DMA,
    ],
    compiler_params=pltpu.CompilerParams(use_tc_tiling_on_sc=False),
)(_user_body)


def kernel(u, i, user_table, item_table):
    it = _renorm_plain(item_table)
    i1 = i.astype(jnp.int32).reshape(-1)
    sout = _item_sc(i1, it)
    ut = _renorm_table(user_table, 10000)
    u1 = u.astype(jnp.int32).reshape(-1)
    out = _user_sc(u1, sout, ut.reshape(_UV, _D))
    return out.reshape(_B, 1)
